# baseline (speedup 1.0000x reference)
import functools
import math

import jax
import jax.numpy as jnp
from jax.experimental import pallas as pl
from jax.experimental.pallas import tpu as pltpu

N = 4096
D = 1024
H = 1024
O = 1024
NB = 8
R = 8
SCALING = 16.0 / 8.0
TILE = 512

_INV_SQRT2 = 1.0 / math.sqrt(2.0)


def _mlp_kernel(xb_ref, fc1_ref, fc1b_ref, fc2_ref, fc2b_ref,
                a1_ref, b1_ref, a2_ref, b2_ref, out_ref):
    xb = xb_ref[:]
    x = xb[:, :D].astype(jnp.bfloat16)
    band = xb[:, D:].astype(jnp.int32)
    lane_band = jax.lax.broadcasted_iota(jnp.int32, (x.shape[0], NB * R), 1) // R
    mask = (lane_band == band).astype(jnp.float32)

    dn_nt = (((1,), (1,)), ((), ()))
    dn_nn = (((1,), (0,)), ((), ()))
    f32 = jnp.float32

    u1 = jax.lax.dot_general(x, a1_ref[:], dn_nn,
                             preferred_element_type=f32) * mask
    h = jax.lax.dot_general(x, fc1_ref[:], dn_nt,
                            preferred_element_type=f32)
    h += fc1b_ref[:]
    h += jax.lax.dot_general(u1.astype(jnp.bfloat16), b1_ref[:], dn_nn,
                             preferred_element_type=f32) * SCALING
    h = 0.5 * h * (1.0 + jax.lax.erf(h * _INV_SQRT2))
    hb = h.astype(jnp.bfloat16)

    u2 = jax.lax.dot_general(hb, a2_ref[:], dn_nn,
                             preferred_element_type=f32) * mask
    out = jax.lax.dot_general(hb, fc2_ref[:], dn_nt,
                              preferred_element_type=f32)
    out += fc2b_ref[:]
    out += jax.lax.dot_general(u2.astype(jnp.bfloat16), b2_ref[:], dn_nn,
                               preferred_element_type=f32) * SCALING
    out_ref[:] = out


@jax.jit
def kernel(x_with_band_info, fc1_w, fc1_b, fc2_w, fc2_b,
           lora_fc1_A, lora_fc1_B, lora_fc2_A, lora_fc2_B):
    bf16 = jnp.bfloat16
    a1f = lora_fc1_A.transpose(1, 0, 2).reshape(D, NB * R).astype(bf16)
    b1f = lora_fc1_B.reshape(NB * R, H).astype(bf16)
    a2f = lora_fc2_A.transpose(1, 0, 2).reshape(H, NB * R).astype(bf16)
    b2f = lora_fc2_B.reshape(NB * R, O).astype(bf16)
    fc1w = fc1_w.astype(bf16)
    fc2w = fc2_w.astype(bf16)
    fc1b = fc1_b.reshape(1, H)
    fc2b = fc2_b.reshape(1, O)

    full = lambda shape: pl.BlockSpec(shape, lambda i: (0, 0))
    grid = (N // TILE,)
    out = pl.pallas_call(
        _mlp_kernel,
        grid=grid,
        in_specs=[
            pl.BlockSpec((TILE, D + 1), lambda i: (i, 0)),
            full((H, D)),
            full((1, H)),
            full((O, H)),
            full((1, O)),
            full((D, NB * R)),
            full((NB * R, H)),
            full((H, NB * R)),
            full((NB * R, O)),
        ],
        out_specs=pl.BlockSpec((TILE, O), lambda i: (i, 0)),
        out_shape=jax.ShapeDtypeStruct((N, O), jnp.float32),
        compiler_params=pltpu.CompilerParams(
            dimension_semantics=("parallel",),
        ),
    )(x_with_band_info, fc1w, fc1b, fc2w, fc2b, a1f, b1f, a2f, b2f)
    return out


# speedup vs baseline: 1.0727x; 1.0727x over previous
import functools
import math

import jax
import jax.numpy as jnp
from jax.experimental import pallas as pl
from jax.experimental.pallas import tpu as pltpu

N = 4096
D = 1024
H = 1024
O = 1024
NB = 8
R = 8
SCALING = 16.0 / 8.0
TILE = 256

_INV_SQRT2 = 1.0 / math.sqrt(2.0)


def _mlp_kernel(xb_ref, fc1_ref, fc1b_ref, fc2_ref, fc2b_ref,
                a1_ref, b1_ref, a2_ref, b2_ref, out_ref):
    xb = xb_ref[:]
    x = xb[:, :D]
    band = xb[:, D:].astype(jnp.int32)
    lane_band = jax.lax.broadcasted_iota(jnp.int32, (x.shape[0], NB * R), 1) // R
    mask = (lane_band == band).astype(jnp.float32)

    dn_nt = (((1,), (1,)), ((), ()))
    dn_nn = (((1,), (0,)), ((), ()))
    f32 = jnp.float32

    u1 = jax.lax.dot_general(x, a1_ref[:], dn_nn,
                             preferred_element_type=f32) * mask
    h = jax.lax.dot_general(x, fc1_ref[:], dn_nt,
                            preferred_element_type=f32)
    h += fc1b_ref[:]
    h += jax.lax.dot_general(u1, b1_ref[:], dn_nn,
                             preferred_element_type=f32) * SCALING
    h = 0.5 * h * (1.0 + jax.lax.erf(h * _INV_SQRT2))

    u2 = jax.lax.dot_general(h, a2_ref[:], dn_nn,
                             preferred_element_type=f32) * mask
    out = jax.lax.dot_general(h, fc2_ref[:], dn_nt,
                              preferred_element_type=f32)
    out += fc2b_ref[:]
    out += jax.lax.dot_general(u2, b2_ref[:], dn_nn,
                               preferred_element_type=f32) * SCALING
    out_ref[:] = out


@jax.jit
def kernel(x_with_band_info, fc1_w, fc1_b, fc2_w, fc2_b,
           lora_fc1_A, lora_fc1_B, lora_fc2_A, lora_fc2_B):
    a1f = lora_fc1_A.transpose(1, 0, 2).reshape(D, NB * R)
    b1f = lora_fc1_B.reshape(NB * R, H)
    a2f = lora_fc2_A.transpose(1, 0, 2).reshape(H, NB * R)
    b2f = lora_fc2_B.reshape(NB * R, O)
    fc1w = fc1_w
    fc2w = fc2_w
    fc1b = fc1_b.reshape(1, H)
    fc2b = fc2_b.reshape(1, O)

    full = lambda shape: pl.BlockSpec(shape, lambda i: (0, 0))
    grid = (N // TILE,)
    out = pl.pallas_call(
        _mlp_kernel,
        grid=grid,
        in_specs=[
            pl.BlockSpec((TILE, D + 1), lambda i: (i, 0)),
            full((H, D)),
            full((1, H)),
            full((O, H)),
            full((1, O)),
            full((D, NB * R)),
            full((NB * R, H)),
            full((H, NB * R)),
            full((NB * R, O)),
        ],
        out_specs=pl.BlockSpec((TILE, O), lambda i: (i, 0)),
        out_shape=jax.ShapeDtypeStruct((N, O), jnp.float32),
        compiler_params=pltpu.CompilerParams(
            dimension_semantics=("parallel",),
        ),
    )(x_with_band_info, fc1w, fc1b, fc2w, fc2b, a1f, b1f, a2f, b2f)
    return out


# speedup vs baseline: 1.1247x; 1.0484x over previous
import functools
import math

import jax
import jax.numpy as jnp
from jax.experimental import pallas as pl
from jax.experimental.pallas import tpu as pltpu

N = 4096
D = 1024
H = 1024
O = 1024
NB = 8
R = 8
SCALING = 16.0 / 8.0
TILE = 1024

_INV_SQRT2 = 1.0 / math.sqrt(2.0)


def _mlp_kernel(xb_ref, fc1_ref, fc1b_ref, fc2_ref, fc2b_ref,
                a1_ref, b1_ref, a2_ref, b2_ref, out_ref):
    xb = xb_ref[:]
    x = xb[:, :D]
    band = xb[:, D:].astype(jnp.int32)
    lane_band = jax.lax.broadcasted_iota(jnp.int32, (x.shape[0], NB * R), 1) // R
    mask = (lane_band == band).astype(jnp.float32)

    dn_nt = (((1,), (1,)), ((), ()))
    dn_nn = (((1,), (0,)), ((), ()))
    f32 = jnp.float32

    u1 = jax.lax.dot_general(x, a1_ref[:], dn_nn,
                             preferred_element_type=f32) * mask
    h = jax.lax.dot_general(x, fc1_ref[:], dn_nt,
                            preferred_element_type=f32)
    h += fc1b_ref[:]
    h += jax.lax.dot_general(u1, b1_ref[:], dn_nn,
                             preferred_element_type=f32) * SCALING
    h = 0.5 * h * (1.0 + jax.lax.erf(h * _INV_SQRT2))

    u2 = jax.lax.dot_general(h, a2_ref[:], dn_nn,
                             preferred_element_type=f32) * mask
    out = jax.lax.dot_general(h, fc2_ref[:], dn_nt,
                              preferred_element_type=f32)
    out += fc2b_ref[:]
    out += jax.lax.dot_general(u2, b2_ref[:], dn_nn,
                               preferred_element_type=f32) * SCALING
    out_ref[:] = out


@jax.jit
def kernel(x_with_band_info, fc1_w, fc1_b, fc2_w, fc2_b,
           lora_fc1_A, lora_fc1_B, lora_fc2_A, lora_fc2_B):
    a1f = lora_fc1_A.transpose(1, 0, 2).reshape(D, NB * R)
    b1f = lora_fc1_B.reshape(NB * R, H)
    a2f = lora_fc2_A.transpose(1, 0, 2).reshape(H, NB * R)
    b2f = lora_fc2_B.reshape(NB * R, O)
    fc1w = fc1_w
    fc2w = fc2_w
    fc1b = fc1_b.reshape(1, H)
    fc2b = fc2_b.reshape(1, O)

    full = lambda shape: pl.BlockSpec(shape, lambda i: (0, 0))
    grid = (N // TILE,)
    out = pl.pallas_call(
        _mlp_kernel,
        grid=grid,
        in_specs=[
            pl.BlockSpec((TILE, D + 1), lambda i: (i, 0)),
            full((H, D)),
            full((1, H)),
            full((O, H)),
            full((1, O)),
            full((D, NB * R)),
            full((NB * R, H)),
            full((H, NB * R)),
            full((NB * R, O)),
        ],
        out_specs=pl.BlockSpec((TILE, O), lambda i: (i, 0)),
        out_shape=jax.ShapeDtypeStruct((N, O), jnp.float32),
        compiler_params=pltpu.CompilerParams(
            dimension_semantics=("parallel",),
        ),
    )(x_with_band_info, fc1w, fc1b, fc2w, fc2b, a1f, b1f, a2f, b2f)
    return out
